# baseline (speedup 1.0000x reference)
import functools

import jax
import jax.numpy as jnp
from jax.experimental import pallas as pl
from jax.experimental.pallas import tpu as pltpu

CODEBOOK_SIZE = 8192
CODEBOOK_DIM = 256
EPSILON = 1e-05

BM = 512
BK = 1024


def _prep_body(es_ref, u_ref, em2_ref, b2_ref):
    em = es_ref[...] / jnp.maximum(u_ref[...], EPSILON)
    b2_ref[...] = jnp.sum(em * em, axis=1)[None, :]
    em2_ref[...] = -2.0 * em


def _local_min_idx(d2, base):
    lmin = jnp.min(d2, axis=1, keepdims=True)
    ids = jax.lax.broadcasted_iota(jnp.int32, (1, BK), 1)
    lidx = jnp.min(
        jnp.where(d2 == lmin, ids, jnp.int32(CODEBOOK_SIZE)),
        axis=1, keepdims=True) + base
    return lmin, lidx


def _fold(lmin, lidx, minval_ref, minidx_ref):
    better = lmin < minval_ref[...]
    minval_ref[...] = jnp.where(better, lmin, minval_ref[...])
    minidx_ref[...] = jnp.where(better, lidx, minidx_ref[...])


def _body(nt, a_ref, em2_ref, b2_ref, o_ref,
          pa_ref, pb_ref, minval_ref, minidx_ref):
    t = pl.program_id(1)
    j0 = 2 * t

    a = a_ref[...]

    pa_ref[...] = jax.lax.dot_general(
        a, em2_ref[pl.ds(j0 * BK, BK), :], (((1,), (1,)), ((), ())),
        preferred_element_type=jnp.float32)

    jprev = jnp.maximum(j0 - 1, 0)
    d2p = b2_ref[:, pl.ds(jprev * BK, BK)] + pb_ref[...]
    lminp, lidxp = _local_min_idx(d2p, jprev * BK)

    @pl.when(t > 0)
    def _fold_prev():
        _fold(lminp, lidxp, minval_ref, minidx_ref)

    pb_ref[...] = jax.lax.dot_general(
        a, em2_ref[pl.ds((j0 + 1) * BK, BK), :], (((1,), (1,)), ((), ())),
        preferred_element_type=jnp.float32)

    d2a = b2_ref[:, pl.ds(j0 * BK, BK)] + pa_ref[...]
    lmina, lidxa = _local_min_idx(d2a, j0 * BK)

    @pl.when(t == 0)
    def _init():
        minval_ref[...] = lmina
        minidx_ref[...] = lidxa

    @pl.when(t > 0)
    def _fold_a():
        _fold(lmina, lidxa, minval_ref, minidx_ref)

    @pl.when(t == nt - 1)
    def _tail():
        d2b = b2_ref[:, pl.ds((j0 + 1) * BK, BK)] + pb_ref[...]
        lminb, lidxb = _local_min_idx(d2b, (j0 + 1) * BK)
        _fold(lminb, lidxb, minval_ref, minidx_ref)
        o_ref[...] = minidx_ref[...]


def kernel(hidden_states, embed_sum, cluster_usage):
    shape = hidden_states.shape
    flat = hidden_states.reshape(-1, shape[-1]).astype(jnp.float32)
    m, d = flat.shape
    kk = embed_sum.shape[0]
    nm = m // BM
    nt = kk // (2 * BK)
    usage = cluster_usage.reshape(kk, 1)

    em2, b2 = pl.pallas_call(
        _prep_body,
        out_shape=(
            jax.ShapeDtypeStruct((kk, d), jnp.float32),
            jax.ShapeDtypeStruct((1, kk), jnp.float32),
        ),
    )(embed_sum, usage)

    out = pl.pallas_call(
        functools.partial(_body, nt),
        grid=(nm, nt),
        in_specs=[
            pl.BlockSpec((BM, d), lambda i, t: (i, 0)),
            pl.BlockSpec((kk, d), lambda i, t: (0, 0)),
            pl.BlockSpec((1, kk), lambda i, t: (0, 0)),
        ],
        out_specs=pl.BlockSpec((BM, 1), lambda i, t: (i, 0)),
        out_shape=jax.ShapeDtypeStruct((m, 1), jnp.int32),
        scratch_shapes=[
            pltpu.VMEM((BM, BK), jnp.float32),
            pltpu.VMEM((BM, BK), jnp.float32),
            pltpu.VMEM((BM, 1), jnp.float32),
            pltpu.VMEM((BM, 1), jnp.int32),
        ],
        compiler_params=pltpu.CompilerParams(
            dimension_semantics=("parallel", "arbitrary")),
    )(flat, em2, b2)
    return out.reshape(shape[:-1])
